# baseline (speedup 1.0000x reference)
import functools

import jax
import jax.numpy as jnp
from jax import lax
from jax.experimental import pallas as pl
from jax.experimental.pallas import tpu as pltpu
from jax.experimental.pallas import tpu_sc as plsc

BATCH = 16384
D_MLP = 256
D_GMF = 64

_NC = 2
_NS = 16
_NW = _NC * _NS
_BPW = BATCH // _NW
_CHUNK = 128
_NCHUNK = _BPW // _CHUNK

_sc_mesh = plsc.VectorSubcoreMesh(core_axis_name="c", subcore_axis_name="s")


@functools.partial(
    pl.kernel,
    mesh=_sc_mesh,
    out_type=[
        jax.ShapeDtypeStruct((BATCH, D_MLP), jnp.float32),
        jax.ShapeDtypeStruct((BATCH, D_MLP), jnp.float32),
    ],
    scratch_types=[
        pltpu.VMEM((_CHUNK,), jnp.int32),
        pltpu.VMEM((_CHUNK,), jnp.int32),
        pltpu.VMEM((_CHUNK, D_MLP), jnp.float32),
        pltpu.VMEM((_CHUNK, D_MLP), jnp.float32),
        pltpu.SemaphoreType.DMA,
    ],
)
def _sc_gather_mlp(users_hbm, items_hbm, uemb_hbm, iemb_hbm, out_u, out_i,
                   uidx_v, iidx_v, urows_v, irows_v, sem):
    wid = lax.axis_index("s") * _NC + lax.axis_index("c")
    base = wid * _BPW

    def chunk_body(k, carry):
        off = base + k * _CHUNK
        pltpu.sync_copy(users_hbm.at[pl.ds(off, _CHUNK)], uidx_v)
        pltpu.sync_copy(items_hbm.at[pl.ds(off, _CHUNK)], iidx_v)
        c1 = pltpu.async_copy(uemb_hbm.at[uidx_v], urows_v, sem)
        c2 = pltpu.async_copy(iemb_hbm.at[iidx_v], irows_v, sem)
        c1.wait()
        c2.wait()
        pltpu.sync_copy(urows_v, out_u.at[pl.ds(off, _CHUNK)])
        pltpu.sync_copy(irows_v, out_i.at[pl.ds(off, _CHUNK)])
        return carry

    lax.fori_loop(0, _NCHUNK, chunk_body, 0)


@functools.partial(
    pl.kernel,
    mesh=_sc_mesh,
    out_type=[
        jax.ShapeDtypeStruct((BATCH, D_GMF), jnp.float32),
    ],
    scratch_types=[
        pltpu.VMEM((_CHUNK,), jnp.int32),
        pltpu.VMEM((_CHUNK,), jnp.int32),
        pltpu.VMEM((_CHUNK, 2 * D_GMF), jnp.float32),
        pltpu.VMEM((_CHUNK, 2 * D_GMF), jnp.float32),
        pltpu.VMEM((_CHUNK, D_GMF), jnp.float32),
        pltpu.SemaphoreType.DMA,
    ],
)
def _sc_gather_gmf(users_hbm, items_hbm, gcat_hbm, out_g,
                   uidx_v, iidx_v, ucat_v, icat_v, g_v, sem):
    wid = lax.axis_index("s") * _NC + lax.axis_index("c")
    base = wid * _BPW

    def chunk_body(k, carry):
        off = base + k * _CHUNK
        pltpu.sync_copy(users_hbm.at[pl.ds(off, _CHUNK)], uidx_v)
        pltpu.sync_copy(items_hbm.at[pl.ds(off, _CHUNK)], iidx_v)
        c1 = pltpu.async_copy(gcat_hbm.at[uidx_v], ucat_v, sem)
        c2 = pltpu.async_copy(gcat_hbm.at[iidx_v], icat_v, sem)
        c1.wait()
        c2.wait()

        def mul_body(r, mc):
            for c in range(D_GMF // 16):
                s = pl.ds(c * 16, 16)
                s_hi = pl.ds(D_GMF + c * 16, 16)
                g_v[r, s] = ucat_v[r, s] * icat_v[r, s_hi]
            return mc

        lax.fori_loop(0, _CHUNK, mul_body, 0)
        pltpu.sync_copy(g_v, out_g.at[pl.ds(off, _CHUNK)])
        return carry

    lax.fori_loop(0, _NCHUNK, chunk_body, 0)


def _concat_body(a_ref, b_ref, o_ref):
    o_ref[:, 0:D_GMF] = a_ref[...]
    o_ref[:, D_GMF:2 * D_GMF] = b_ref[...]


def _concat_gmf(a, b, block_m=10000):
    n = a.shape[0]
    grid = (pl.cdiv(n, block_m),)
    return pl.pallas_call(
        _concat_body,
        grid=grid,
        in_specs=[
            pl.BlockSpec((block_m, D_GMF), lambda m: (m, 0)),
            pl.BlockSpec((block_m, D_GMF), lambda m: (m, 0)),
        ],
        out_specs=pl.BlockSpec((block_m, 2 * D_GMF), lambda m: (m, 0)),
        out_shape=jax.ShapeDtypeStruct((n, 2 * D_GMF), jnp.float32),
        compiler_params=pltpu.CompilerParams(
            dimension_semantics=("arbitrary",),
        ),
    )(a, b)


def _dense_body(u_ref, i_ref, g_ref, w0a_ref, w0b_ref, b0_ref, w1_ref,
                b1_ref, w2_ref, b2_ref, pwa_ref, pwb_ref, pb_ref, o_ref):
    h = jnp.dot(u_ref[...], w0a_ref[...], preferred_element_type=jnp.float32)
    h += jnp.dot(i_ref[...], w0b_ref[...], preferred_element_type=jnp.float32)
    h = jnp.maximum(h + b0_ref[...], 0.0)
    h = jnp.dot(h, w1_ref[...], preferred_element_type=jnp.float32)
    h = jnp.maximum(h + b1_ref[...], 0.0)
    h = jnp.dot(h, w2_ref[...], preferred_element_type=jnp.float32)
    h = jnp.maximum(h + b2_ref[...], 0.0)
    logit = jnp.dot(h, pwa_ref[...], preferred_element_type=jnp.float32)
    logit += jnp.dot(g_ref[...], pwb_ref[...], preferred_element_type=jnp.float32)
    logit += pb_ref[0, 0]
    o_ref[...] = 1.0 / (1.0 + jnp.exp(-logit))


def _dense(u_rows, i_rows, g, w0a, w0b, b0, w1, b1, w2, b2, pwa, pwb, pb,
           block_m=1024):
    grid = (BATCH // block_m,)
    full = lambda m: (0, 0)
    return pl.pallas_call(
        _dense_body,
        grid=grid,
        in_specs=[
            pl.BlockSpec((block_m, D_MLP), lambda m: (m, 0)),
            pl.BlockSpec((block_m, D_MLP), lambda m: (m, 0)),
            pl.BlockSpec((block_m, D_GMF), lambda m: (m, 0)),
            pl.BlockSpec((D_MLP, 256), full),
            pl.BlockSpec((D_MLP, 256), full),
            pl.BlockSpec((1, 256), full),
            pl.BlockSpec((256, 128), full),
            pl.BlockSpec((1, 128), full),
            pl.BlockSpec((128, 64), full),
            pl.BlockSpec((1, 64), full),
            pl.BlockSpec((64, 1), full),
            pl.BlockSpec((64, 1), full),
            pl.BlockSpec((1, 1), full),
        ],
        out_specs=pl.BlockSpec((block_m, 1), lambda m: (m, 0)),
        out_shape=jax.ShapeDtypeStruct((BATCH, 1), jnp.float32),
        compiler_params=pltpu.CompilerParams(
            dimension_semantics=("arbitrary",),
        ),
    )(u_rows, i_rows, g, w0a, w0b, b0, w1, b1, w2, b2, pwa, pwb, pb)


def kernel(users, items, user_emb_mlp, item_emb_mlp, user_emb_gmf,
           item_emb_gmf, mlp_w0, mlp_b0, mlp_w1, mlp_b1, mlp_w2, mlp_b2,
           pred_w, pred_b):
    users = users.astype(jnp.int32)
    items = items.astype(jnp.int32)

    gmf_cat = _concat_gmf(user_emb_gmf, item_emb_gmf)
    u_rows, i_rows = _sc_gather_mlp(users, items, user_emb_mlp, item_emb_mlp)
    (g,) = _sc_gather_gmf(users, items, gmf_cat)

    w0a = mlp_w0[:D_MLP]
    w0b = mlp_w0[D_MLP:]
    pwa = pred_w[:D_GMF]
    pwb = pred_w[D_GMF:]
    out = _dense(u_rows, i_rows, g, w0a, w0b, mlp_b0.reshape(1, -1),
                 mlp_w1, mlp_b1.reshape(1, -1), mlp_w2,
                 mlp_b2.reshape(1, -1), pwa, pwb, pred_b.reshape(1, 1))
    return out.reshape(-1)


# speedup vs baseline: 1.1578x; 1.1578x over previous
import functools

import jax
import jax.numpy as jnp
from jax import lax
from jax.experimental import pallas as pl
from jax.experimental.pallas import tpu as pltpu
from jax.experimental.pallas import tpu_sc as plsc

BATCH = 16384
D_MLP = 256
D_GMF = 64

_NC = 2
_NS = 16
_NW = _NC * _NS
_BPW = BATCH // _NW
_CHUNK = 64
_NCHUNK = _BPW // _CHUNK

_sc_mesh = plsc.VectorSubcoreMesh(core_axis_name="c", subcore_axis_name="s")


@functools.partial(
    pl.kernel,
    mesh=_sc_mesh,
    out_type=[
        jax.ShapeDtypeStruct((BATCH, D_MLP), jnp.float32),
        jax.ShapeDtypeStruct((BATCH, D_MLP), jnp.float32),
        jax.ShapeDtypeStruct((BATCH, D_GMF), jnp.float32),
    ],
    scratch_types=[
        pltpu.VMEM((_BPW,), jnp.int32),
        pltpu.VMEM((_BPW,), jnp.int32),
        pltpu.VMEM((2, _CHUNK, D_MLP), jnp.float32),
        pltpu.VMEM((2, _CHUNK, D_MLP), jnp.float32),
        pltpu.VMEM((2, _CHUNK, 2 * D_GMF), jnp.float32),
        pltpu.VMEM((2, _CHUNK, 2 * D_GMF), jnp.float32),
        pltpu.VMEM((_CHUNK, D_GMF), jnp.float32),
        pltpu.SemaphoreType.DMA,
        pltpu.SemaphoreType.DMA,
    ],
)
def _sc_gather(users_hbm, items_hbm, uemb_hbm, iemb_hbm, gcat_hbm,
               out_u, out_i, out_g,
               uidx_v, iidx_v, urows_v, irows_v, ucat_v, icat_v, g_v,
               sem0, sem1):
    wid = lax.axis_index("s") * _NC + lax.axis_index("c")
    base = wid * _BPW
    sems = (sem0, sem1)

    pltpu.sync_copy(users_hbm.at[pl.ds(base, _BPW)], uidx_v)
    pltpu.sync_copy(items_hbm.at[pl.ds(base, _BPW)], iidx_v)

    def fire(k):
        p = k % 2
        uix = uidx_v.at[pl.ds(k * _CHUNK, _CHUNK)]
        iix = iidx_v.at[pl.ds(k * _CHUNK, _CHUNK)]
        return (
            pltpu.async_copy(uemb_hbm.at[uix], urows_v.at[p], sems[p]),
            pltpu.async_copy(iemb_hbm.at[iix], irows_v.at[p], sems[p]),
            pltpu.async_copy(gcat_hbm.at[uix], ucat_v.at[p], sems[p]),
            pltpu.async_copy(gcat_hbm.at[iix], icat_v.at[p], sems[p]),
        )

    inflight = fire(0)
    for k in range(_NCHUNK):
        nxt = fire(k + 1) if k + 1 < _NCHUNK else None
        for c in inflight:
            c.wait()
        p = k % 2
        off = base + k * _CHUNK

        def mul_body(r, mc):
            for c in range(D_GMF // 16):
                s = pl.ds(c * 16, 16)
                s_hi = pl.ds(D_GMF + c * 16, 16)
                g_v[r, s] = ucat_v[p, r, s] * icat_v[p, r, s_hi]
            return mc

        lax.fori_loop(0, _CHUNK, mul_body, 0)

        pltpu.sync_copy(urows_v.at[p], out_u.at[pl.ds(off, _CHUNK)])
        pltpu.sync_copy(irows_v.at[p], out_i.at[pl.ds(off, _CHUNK)])
        pltpu.sync_copy(g_v, out_g.at[pl.ds(off, _CHUNK)])
        inflight = nxt


def _dense_body(u_ref, i_ref, g_ref, w0a_ref, w0b_ref, b0_ref, w1_ref,
                b1_ref, w2_ref, b2_ref, pwa_ref, pwb_ref, pb_ref, o_ref):
    h = jnp.dot(u_ref[...], w0a_ref[...], preferred_element_type=jnp.float32)
    h += jnp.dot(i_ref[...], w0b_ref[...], preferred_element_type=jnp.float32)
    h = jnp.maximum(h + b0_ref[...], 0.0)
    h = jnp.dot(h, w1_ref[...], preferred_element_type=jnp.float32)
    h = jnp.maximum(h + b1_ref[...], 0.0)
    h = jnp.dot(h, w2_ref[...], preferred_element_type=jnp.float32)
    h = jnp.maximum(h + b2_ref[...], 0.0)
    logit = jnp.dot(h, pwa_ref[...], preferred_element_type=jnp.float32)
    logit += jnp.dot(g_ref[...], pwb_ref[...], preferred_element_type=jnp.float32)
    logit += pb_ref[0, 0]
    o_ref[...] = 1.0 / (1.0 + jnp.exp(-logit))


def _dense(u_rows, i_rows, g, w0a, w0b, b0, w1, b1, w2, b2, pwa, pwb, pb,
           block_m=2048):
    grid = (BATCH // block_m,)
    full = lambda m: (0, 0)
    return pl.pallas_call(
        _dense_body,
        grid=grid,
        in_specs=[
            pl.BlockSpec((block_m, D_MLP), lambda m: (m, 0)),
            pl.BlockSpec((block_m, D_MLP), lambda m: (m, 0)),
            pl.BlockSpec((block_m, D_GMF), lambda m: (m, 0)),
            pl.BlockSpec((D_MLP, 256), full),
            pl.BlockSpec((D_MLP, 256), full),
            pl.BlockSpec((1, 256), full),
            pl.BlockSpec((256, 128), full),
            pl.BlockSpec((1, 128), full),
            pl.BlockSpec((128, 64), full),
            pl.BlockSpec((1, 64), full),
            pl.BlockSpec((64, 1), full),
            pl.BlockSpec((64, 1), full),
            pl.BlockSpec((1, 1), full),
        ],
        out_specs=pl.BlockSpec((block_m, 1), lambda m: (m, 0)),
        out_shape=jax.ShapeDtypeStruct((BATCH, 1), jnp.float32),
        compiler_params=pltpu.CompilerParams(
            dimension_semantics=("arbitrary",),
        ),
    )(u_rows, i_rows, g, w0a, w0b, b0, w1, b1, w2, b2, pwa, pwb, pb)


def kernel(users, items, user_emb_mlp, item_emb_mlp, user_emb_gmf,
           item_emb_gmf, mlp_w0, mlp_b0, mlp_w1, mlp_b1, mlp_w2, mlp_b2,
           pred_w, pred_b):
    users = users.astype(jnp.int32)
    items = items.astype(jnp.int32)

    gmf_cat = jnp.concatenate([user_emb_gmf, item_emb_gmf], axis=1)
    u_rows, i_rows, g = _sc_gather(users, items, user_emb_mlp, item_emb_mlp,
                                   gmf_cat)

    w0a = mlp_w0[:D_MLP]
    w0b = mlp_w0[D_MLP:]
    pwa = pred_w[:D_GMF]
    pwb = pred_w[D_GMF:]
    out = _dense(u_rows, i_rows, g, w0a, w0b, mlp_b0.reshape(1, -1),
                 mlp_w1, mlp_b1.reshape(1, -1), mlp_w2,
                 mlp_b2.reshape(1, -1), pwa, pwb, pred_b.reshape(1, 1))
    return out.reshape(-1)
